# baseline (speedup 1.0000x reference)
import jax
import jax.numpy as jnp
from jax.experimental import pallas as pl
from jax.experimental.pallas import tpu as pltpu

N = 2048
F_IN = 512
F_OUT = 8
BLOCK_ROWS = 1024
ALPHA = 0.2


def _gat_kernel(x_ref, adj_ref, w_ref, a_ref, out_ref, h9_ref, bd_ref, r_ref):
    i = pl.program_id(0)

    @pl.when(i == 0)
    def _():
        h = jnp.dot(x_ref[...], w_ref[...], preferred_element_type=jnp.float32)
        ones = jnp.ones((N, 1), dtype=jnp.float32)
        zeros = jnp.zeros((N, 7), dtype=jnp.float32)
        h9_ref[...] = jnp.concatenate([h, ones, zeros], axis=1).astype(jnp.bfloat16)
        a_src = a_ref[0, :F_OUT].reshape(F_OUT, 1)
        a_dst = a_ref[0, F_OUT:].reshape(F_OUT, 1)
        s = jnp.dot(h, a_src, preferred_element_type=jnp.float32)
        d = jnp.dot(h, a_dst, preferred_element_type=jnp.float32)
        r_ref[...] = jnp.exp((1.0 - ALPHA) * s).astype(jnp.bfloat16)
        d_row = d.reshape(1, N)
        bd_ref[...] = jnp.concatenate(
            [jnp.exp(-d_row), jnp.exp(-ALPHA * d_row)], axis=0
        ).astype(jnp.bfloat16)

    r = r_ref[pl.ds(i * BLOCK_ROWS, BLOCK_ROWS), 0:1]
    B = bd_ref[0:1, :]
    D = bd_ref[1:2, :]
    w = jnp.minimum(B, r * D)
    adj16 = adj_ref[...].astype(jnp.int16)
    ebits = adj16 * jax.lax.bitcast_convert_type(w, jnp.int16)
    e = jax.lax.bitcast_convert_type(ebits, jnp.bfloat16)
    agg = jnp.dot(e, h9_ref[...], preferred_element_type=jnp.float32)
    v = agg[:, :F_OUT] / agg[:, F_OUT : F_OUT + 1]
    out_ref[...] = jnp.where(v > 0, v, jnp.exp(jnp.minimum(v, 0.0)) - 1.0)


@jax.jit
def kernel(input, adj, W, a):
    grid = N // BLOCK_ROWS
    return pl.pallas_call(
        _gat_kernel,
        grid=(grid,),
        in_specs=[
            pl.BlockSpec((N, F_IN), lambda i: (0, 0)),
            pl.BlockSpec((BLOCK_ROWS, N), lambda i: (i, 0)),
            pl.BlockSpec((F_IN, F_OUT), lambda i: (0, 0)),
            pl.BlockSpec((1, 2 * F_OUT), lambda i: (0, 0)),
        ],
        out_specs=pl.BlockSpec((BLOCK_ROWS, F_OUT), lambda i: (i, 0)),
        out_shape=jax.ShapeDtypeStruct((N, F_OUT), jnp.float32),
        compiler_params=pltpu.CompilerParams(vmem_limit_bytes=120 * 1024 * 1024),
        scratch_shapes=[
            pltpu.VMEM((N, 2 * F_OUT), jnp.bfloat16),
            pltpu.VMEM((2, N), jnp.bfloat16),
            pltpu.VMEM((N, 1), jnp.bfloat16),
        ],
    )(input, adj, W, a)


# speedup vs baseline: 1.0211x; 1.0211x over previous
import jax
import jax.numpy as jnp
from jax.experimental import pallas as pl
from jax.experimental.pallas import tpu as pltpu

N = 2048
F_IN = 512
F_OUT = 8
BLOCK_ROWS = 1024
ALPHA = 0.2


def _gat_kernel(x_ref, adj_ref, w_ref, a_ref, out_ref, h9_ref, bd_ref, r_ref):
    i = pl.program_id(0)

    @pl.when(i == 0)
    def _():
        h = jnp.dot(x_ref[...], w_ref[...], preferred_element_type=jnp.float32)
        ones = jnp.ones((N, 1), dtype=jnp.float32)
        zeros = jnp.zeros((N, 7), dtype=jnp.float32)
        h9_ref[...] = jnp.concatenate([h, ones, zeros], axis=1).astype(jnp.bfloat16)
        a_src = a_ref[0, :F_OUT].reshape(F_OUT, 1)
        a_dst = a_ref[0:1, F_OUT:]
        s = jnp.dot(h, a_src, preferred_element_type=jnp.float32)
        d_row = jax.lax.dot_general(
            a_dst, h, (((1,), (1,)), ((), ())),
            preferred_element_type=jnp.float32,
        )
        r_ref[...] = jnp.exp((1.0 - ALPHA) * s).astype(jnp.bfloat16)
        bd_ref[...] = jnp.concatenate(
            [jnp.exp(-d_row), jnp.exp(-ALPHA * d_row)], axis=0
        ).astype(jnp.bfloat16)

    r = r_ref[pl.ds(i * BLOCK_ROWS, BLOCK_ROWS), 0:1]
    B = bd_ref[0:1, :]
    D = bd_ref[1:2, :]
    w = jnp.minimum(B, r * D)
    adj16 = adj_ref[...].astype(jnp.int16)
    ebits = adj16 * jax.lax.bitcast_convert_type(w, jnp.int16)
    e = jax.lax.bitcast_convert_type(ebits, jnp.bfloat16)
    agg = jnp.dot(e, h9_ref[...], preferred_element_type=jnp.float32)
    v = agg[:, :F_OUT] / agg[:, F_OUT : F_OUT + 1]
    out_ref[...] = jnp.where(v > 0, v, jnp.exp(jnp.minimum(v, 0.0)) - 1.0)


@jax.jit
def kernel(input, adj, W, a):
    grid = N // BLOCK_ROWS
    return pl.pallas_call(
        _gat_kernel,
        grid=(grid,),
        in_specs=[
            pl.BlockSpec((N, F_IN), lambda i: (0, 0)),
            pl.BlockSpec((BLOCK_ROWS, N), lambda i: (i, 0)),
            pl.BlockSpec((F_IN, F_OUT), lambda i: (0, 0)),
            pl.BlockSpec((1, 2 * F_OUT), lambda i: (0, 0)),
        ],
        out_specs=pl.BlockSpec((BLOCK_ROWS, F_OUT), lambda i: (i, 0)),
        out_shape=jax.ShapeDtypeStruct((N, F_OUT), jnp.float32),
        compiler_params=pltpu.CompilerParams(vmem_limit_bytes=120 * 1024 * 1024),
        scratch_shapes=[
            pltpu.VMEM((N, 2 * F_OUT), jnp.bfloat16),
            pltpu.VMEM((2, N), jnp.bfloat16),
            pltpu.VMEM((N, 1), jnp.bfloat16),
        ],
    )(input, adj, W, a)
